# baseline (speedup 1.0000x reference)
import jax
import jax.numpy as jnp
from jax import lax
from jax.experimental import pallas as pl
from jax.experimental.pallas import tpu as pltpu
from jax.experimental.pallas import tpu_sc as plsc

NN = 10000
EE = 320000
NW = 32
CH = 128
NCH = 80
BCH = 16
NBLK = NCH // BCH
EPAD = NW * NCH * CH
NPAD = 10112
STRIPE = NPAD // 16

_SC_MESH = plsc.VectorSubcoreMesh(core_axis_name="c", subcore_axis_name="s")


def _sc_agg_body(p_hbm, src_hbm, dst_hbm, z_hbm, agg_out,
                 idx_s, idx_d, rows, acc_sh, sem0, sem1):
  c = lax.axis_index("c")
  s = lax.axis_index("s")
  w = c * 16 + s
  sems = (sem0, sem1)
  pltpu.sync_copy(z_hbm, acc_sh.at[pl.ds(s * STRIPE, STRIPE)])
  plsc.subcore_barrier()

  def start(j, buf):
    pltpu.async_copy(p_hbm.at[idx_s.at[j, pl.ds(0, 64)]],
                     rows.at[buf, pl.ds(0, 64)], sems[buf])
    pltpu.async_copy(p_hbm.at[idx_s.at[j, pl.ds(64, 64)]],
                     rows.at[buf, pl.ds(64, 64)], sems[buf])

  def finish(j, buf):
    pltpu.make_async_copy(p_hbm.at[idx_s.at[j, pl.ds(0, 64)]],
                          rows.at[buf, pl.ds(0, 64)], sems[buf]).wait()
    pltpu.make_async_copy(p_hbm.at[idx_s.at[j, pl.ds(64, 64)]],
                          rows.at[buf, pl.ds(64, 64)], sems[buf]).wait()
    pltpu.sync_copy(rows.at[buf], acc_sh.at[idx_d.at[j]], add=True)

  def blk(b, carry):
    pltpu.sync_copy(src_hbm.at[w, pl.ds(b * BCH, BCH)], idx_s)
    pltpu.sync_copy(dst_hbm.at[w, pl.ds(b * BCH, BCH)], idx_d)
    start(0, 0)

    def pair(k, carry2):
      j0 = 2 * k
      start(j0 + 1, 1)
      finish(j0, 0)

      @pl.when(j0 + 2 < BCH)
      def _():
        start(j0 + 2, 0)

      finish(j0 + 1, 1)
      return carry2

    lax.fori_loop(0, BCH // 2, pair, 0)
    return carry

  lax.fori_loop(0, NBLK, blk, 0)
  plsc.subcore_barrier()
  pltpu.sync_copy(acc_sh.at[pl.ds(s * STRIPE, STRIPE)],
                  agg_out.at[c, pl.ds(s * STRIPE, STRIPE)])


_sc_agg = pl.kernel(
    _sc_agg_body,
    out_type=[jax.ShapeDtypeStruct((2, NPAD, 128), jnp.float32)],
    mesh=_SC_MESH,
    scratch_types=[
        pltpu.VMEM((BCH, CH), jnp.int32),
        pltpu.VMEM((BCH, CH), jnp.int32),
        pltpu.VMEM((2, CH, 128), jnp.float32),
        pltpu.VMEM_SHARED((NPAD, 128), jnp.float32),
        pltpu.SemaphoreType.DMA,
        pltpu.SemaphoreType.DMA,
    ])


def _sc_deg_body(dst_hbm, z_hbm, ones_hbm, deg_out, idx_d, ones_v, acc_sh):
  c = lax.axis_index("c")
  s = lax.axis_index("s")
  w = c * 16 + s
  pltpu.sync_copy(z_hbm, acc_sh.at[pl.ds(s * STRIPE, STRIPE)])
  pltpu.sync_copy(ones_hbm, ones_v)
  plsc.subcore_barrier()

  def blk(b, carry):
    pltpu.sync_copy(dst_hbm.at[w, pl.ds(b * BCH, BCH)], idx_d)

    def chunk(j, carry2):
      pltpu.sync_copy(ones_v, acc_sh.at[idx_d.at[j]], add=True)
      return carry2

    lax.fori_loop(0, BCH, chunk, 0)
    return carry

  lax.fori_loop(0, NBLK, blk, 0)
  plsc.subcore_barrier()
  pltpu.sync_copy(acc_sh.at[pl.ds(s * STRIPE, STRIPE)],
                  deg_out.at[c, pl.ds(s * STRIPE, STRIPE)])


_sc_deg = pl.kernel(
    _sc_deg_body,
    out_type=[jax.ShapeDtypeStruct((2, NPAD, 128), jnp.float32)],
    mesh=_SC_MESH,
    scratch_types=[
        pltpu.VMEM((BCH, CH), jnp.int32),
        pltpu.VMEM((CH, 128), jnp.float32),
        pltpu.VMEM_SHARED((NPAD, 128), jnp.float32),
    ])

_R = 400


def _mm_body(x_ref, w_ref, o_ref):
  o_ref[...] = jnp.dot(x_ref[...], w_ref[...],
                       preferred_element_type=jnp.float32)


def _mm(x, w):
  n, k = x.shape
  m = w.shape[1]
  return pl.pallas_call(
      _mm_body,
      grid=(n // _R,),
      in_specs=[pl.BlockSpec((_R, k), lambda i: (i, 0)),
                pl.BlockSpec((k, m), lambda i: (0, 0))],
      out_specs=pl.BlockSpec((_R, m), lambda i: (i, 0)),
      out_shape=jax.ShapeDtypeStruct((n, m), jnp.float32),
  )(x, w)


def _combine_body(x_ref, agg_ref, deg_ref, ws_ref, b_ref, wn_ref,
                  h_ref, p_ref):
  aggv = agg_ref[0] + agg_ref[1]
  degv = deg_ref[0, :, 0:1] + deg_ref[1, :, 0:1]
  r = 1.0 / (degv + 1.0)
  h = jnp.maximum(
      jnp.dot(x_ref[...], ws_ref[...], preferred_element_type=jnp.float32)
      + aggv * r + b_ref[...], 0.0)
  h_ref[...] = h
  if p_ref is not None:
    p_ref[...] = jnp.dot(h, wn_ref[...], preferred_element_type=jnp.float32)


def _combine(x, agg, deg, w_self, b, w_next=None):
  in_specs = [
      pl.BlockSpec((_R, 128), lambda i: (i, 0)),
      pl.BlockSpec((2, _R, 128), lambda i: (0, i, 0)),
      pl.BlockSpec((2, _R, 128), lambda i: (0, i, 0)),
      pl.BlockSpec((128, 128), lambda i: (0, 0)),
      pl.BlockSpec((1, 128), lambda i: (0, 0)),
  ]
  out_specs = [pl.BlockSpec((_R, 128), lambda i: (i, 0))]
  out_shape = [jax.ShapeDtypeStruct((NN, 128), jnp.float32)]
  args = [x, agg, deg, w_self, b]
  if w_next is not None:
    dn = w_next.shape[1]
    in_specs.append(pl.BlockSpec((128, dn), lambda i: (0, 0)))
    out_specs.append(pl.BlockSpec((_R, dn), lambda i: (i, 0)))
    out_shape.append(jax.ShapeDtypeStruct((NN, dn), jnp.float32))
    args.append(w_next)
    body = _combine_body
  else:
    body = lambda x_ref, agg_ref, deg_ref, ws_ref, b_ref, h_ref: (
        _combine_body(x_ref, agg_ref, deg_ref, ws_ref, b_ref, None,
                      h_ref, None))
  return pl.pallas_call(
      body,
      grid=(NN // _R,),
      in_specs=in_specs,
      out_specs=out_specs,
      out_shape=out_shape,
  )(*args)


def _final_body(x_ref, agg_ref, deg_ref, ws_ref, wn_ref, b_ref, o_ref):
  aggv = agg_ref[0] + agg_ref[1]
  degv = deg_ref[0, :, 0:1] + deg_ref[1, :, 0:1]
  hn = aggv * (1.0 / (degv + 1.0))
  o_ref[...] = (
      jnp.dot(x_ref[...], ws_ref[...], preferred_element_type=jnp.float32)
      + jnp.dot(hn, wn_ref[...], preferred_element_type=jnp.float32)
      + b_ref[...])


def _final(x, agg, deg, w_self, w_neigh, b):
  dn = w_self.shape[1]
  return pl.pallas_call(
      _final_body,
      grid=(NN // _R,),
      in_specs=[
          pl.BlockSpec((_R, 128), lambda i: (i, 0)),
          pl.BlockSpec((2, _R, 128), lambda i: (0, i, 0)),
          pl.BlockSpec((2, _R, 128), lambda i: (0, i, 0)),
          pl.BlockSpec((128, dn), lambda i: (0, 0)),
          pl.BlockSpec((128, dn), lambda i: (0, 0)),
          pl.BlockSpec((1, dn), lambda i: (0, 0)),
      ],
      out_specs=pl.BlockSpec((_R, dn), lambda i: (i, 0)),
      out_shape=jax.ShapeDtypeStruct((NN, dn), jnp.float32),
  )(x, agg, deg, w_self, w_neigh, b)


def kernel(features, edge_index, W_self1, W_neigh1, b1,
           W_self2, W_neigh2, b2, W_self3, W_neigh3, b3):
  src = edge_index[0].astype(jnp.int32)
  dst = edge_index[1].astype(jnp.int32)
  npad = EPAD - EE
  srcp = jnp.concatenate([src, jnp.zeros((npad,), jnp.int32)]).reshape(
      NW, NCH, CH)
  dstp = jnp.concatenate([dst, jnp.full((npad,), NN, jnp.int32)]).reshape(
      NW, NCH, CH)
  z128 = jnp.zeros((STRIPE, 128), jnp.float32)
  ones = jnp.ones((CH, 128), jnp.float32)

  deg, = _sc_deg(dstp, z128, ones)
  p1 = _mm(features, W_neigh1)
  agg1, = _sc_agg(p1, srcp, dstp, z128)
  h1, p2 = _combine(features, agg1, deg, W_self1,
                    b1.reshape(1, -1), W_neigh2)
  agg2, = _sc_agg(p2, srcp, dstp, z128)
  h2, = _combine(h1, agg2, deg, W_self2, b2.reshape(1, -1))
  agg3, = _sc_agg(h2, srcp, dstp, z128)
  return _final(h2, agg3, deg, W_self3, W_neigh3, b3.reshape(1, -1))


# speedup vs baseline: 1.0288x; 1.0288x over previous
import jax
import jax.numpy as jnp
from jax import lax
from jax.experimental import pallas as pl
from jax.experimental.pallas import tpu as pltpu
from jax.experimental.pallas import tpu_sc as plsc

NN = 10000
EE = 320000
NW = 32
CH = 128
NCH = 80
TOTCH = NW * NCH
BCH = 8
CH0 = 40
CH1 = 120
EPAD = TOTCH * CH
NPAD = 10112
STRIPE = NPAD // 16
DEG_BCH = 16
DEG_NBLK = NCH // DEG_BCH

_SC_MESH = plsc.VectorSubcoreMesh(core_axis_name="c", subcore_axis_name="s")


def _sc_agg_body(p_hbm, src_hbm, dst_hbm, z_hbm, agg_out,
                 idx_s, idx_d, rows, acc_sh, sem0, sem1):
  c = lax.axis_index("c")
  s = lax.axis_index("s")
  sems = (sem0, sem1)
  pltpu.sync_copy(z_hbm, acc_sh.at[pl.ds(s * STRIPE, STRIPE)])
  plsc.subcore_barrier()
  base = CH0 * s + c * (16 * CH0 + (CH1 - CH0) * s)
  nblk = (CH0 // BCH) + c * ((CH1 - CH0) // BCH)

  def start(j, buf):
    pltpu.async_copy(p_hbm.at[idx_s.at[j]], rows.at[buf], sems[buf])

  def finish(j, buf):
    pltpu.make_async_copy(p_hbm.at[idx_s.at[j]], rows.at[buf],
                          sems[buf]).wait()
    pltpu.sync_copy(rows.at[buf], acc_sh.at[idx_d.at[j]], add=True)

  def blk(b, carry):
    r0 = pl.multiple_of(base + b * BCH, 8)
    pltpu.sync_copy(src_hbm.at[pl.ds(r0, BCH)], idx_s)
    pltpu.sync_copy(dst_hbm.at[pl.ds(r0, BCH)], idx_d)
    start(0, 0)

    def pair(k, carry2):
      j0 = 2 * k
      start(j0 + 1, 1)
      finish(j0, 0)

      @pl.when(j0 + 2 < BCH)
      def _():
        start(j0 + 2, 0)

      finish(j0 + 1, 1)
      return carry2

    lax.fori_loop(0, BCH // 2, pair, 0)
    return carry

  lax.fori_loop(0, nblk, blk, 0)
  plsc.subcore_barrier()
  pltpu.sync_copy(acc_sh.at[pl.ds(s * STRIPE, STRIPE)],
                  agg_out.at[c, pl.ds(s * STRIPE, STRIPE)])


_sc_agg = pl.kernel(
    _sc_agg_body,
    out_type=[jax.ShapeDtypeStruct((2, NPAD, 128), jnp.float32)],
    mesh=_SC_MESH,
    scratch_types=[
        pltpu.VMEM((BCH, CH), jnp.int32),
        pltpu.VMEM((BCH, CH), jnp.int32),
        pltpu.VMEM((2, CH, 128), jnp.float32),
        pltpu.VMEM_SHARED((NPAD, 128), jnp.float32),
        pltpu.SemaphoreType.DMA,
        pltpu.SemaphoreType.DMA,
    ])


def _sc_deg_body(dst_hbm, z_hbm, ones_hbm, deg_out, idx_d, ones_v, acc_sh):
  c = lax.axis_index("c")
  s = lax.axis_index("s")
  w = c * 16 + s
  pltpu.sync_copy(z_hbm, acc_sh.at[pl.ds(s * STRIPE, STRIPE)])
  pltpu.sync_copy(ones_hbm, ones_v)
  plsc.subcore_barrier()

  def blk(b, carry):
    pltpu.sync_copy(dst_hbm.at[w, pl.ds(b * DEG_BCH, DEG_BCH)], idx_d)

    def chunk(j, carry2):
      pltpu.sync_copy(ones_v, acc_sh.at[idx_d.at[j]], add=True)
      return carry2

    lax.fori_loop(0, DEG_BCH, chunk, 0)
    return carry

  lax.fori_loop(0, DEG_NBLK, blk, 0)
  plsc.subcore_barrier()
  pltpu.sync_copy(acc_sh.at[pl.ds(s * STRIPE, STRIPE)],
                  deg_out.at[c, pl.ds(s * STRIPE, STRIPE)])


_sc_deg = pl.kernel(
    _sc_deg_body,
    out_type=[jax.ShapeDtypeStruct((2, NPAD, 128), jnp.float32)],
    mesh=_SC_MESH,
    scratch_types=[
        pltpu.VMEM((DEG_BCH, CH), jnp.int32),
        pltpu.VMEM((CH, 128), jnp.float32),
        pltpu.VMEM_SHARED((NPAD, 128), jnp.float32),
    ])

_R = 400


def _mm_body(x_ref, w_ref, o_ref):
  o_ref[...] = jnp.dot(x_ref[...], w_ref[...],
                       preferred_element_type=jnp.float32)


def _mm(x, w):
  n, k = x.shape
  m = w.shape[1]
  return pl.pallas_call(
      _mm_body,
      grid=(n // _R,),
      in_specs=[pl.BlockSpec((_R, k), lambda i: (i, 0)),
                pl.BlockSpec((k, m), lambda i: (0, 0))],
      out_specs=pl.BlockSpec((_R, m), lambda i: (i, 0)),
      out_shape=jax.ShapeDtypeStruct((n, m), jnp.float32),
  )(x, w)


def _combine_body(x_ref, agg_ref, deg_ref, ws_ref, b_ref, wn_ref,
                  h_ref, p_ref):
  aggv = agg_ref[0] + agg_ref[1]
  degv = deg_ref[0, :, 0:1] + deg_ref[1, :, 0:1]
  r = 1.0 / (degv + 1.0)
  h = jnp.maximum(
      jnp.dot(x_ref[...], ws_ref[...], preferred_element_type=jnp.float32)
      + aggv * r + b_ref[...], 0.0)
  h_ref[...] = h
  if p_ref is not None:
    p_ref[...] = jnp.dot(h, wn_ref[...], preferred_element_type=jnp.float32)


def _combine(x, agg, deg, w_self, b, w_next=None):
  in_specs = [
      pl.BlockSpec((_R, 128), lambda i: (i, 0)),
      pl.BlockSpec((2, _R, 128), lambda i: (0, i, 0)),
      pl.BlockSpec((2, _R, 128), lambda i: (0, i, 0)),
      pl.BlockSpec((128, 128), lambda i: (0, 0)),
      pl.BlockSpec((1, 128), lambda i: (0, 0)),
  ]
  out_specs = [pl.BlockSpec((_R, 128), lambda i: (i, 0))]
  out_shape = [jax.ShapeDtypeStruct((NN, 128), jnp.float32)]
  args = [x, agg, deg, w_self, b]
  if w_next is not None:
    dn = w_next.shape[1]
    in_specs.append(pl.BlockSpec((128, dn), lambda i: (0, 0)))
    out_specs.append(pl.BlockSpec((_R, dn), lambda i: (i, 0)))
    out_shape.append(jax.ShapeDtypeStruct((NN, dn), jnp.float32))
    args.append(w_next)
    body = _combine_body
  else:
    body = lambda x_ref, agg_ref, deg_ref, ws_ref, b_ref, h_ref: (
        _combine_body(x_ref, agg_ref, deg_ref, ws_ref, b_ref, None,
                      h_ref, None))
  return pl.pallas_call(
      body,
      grid=(NN // _R,),
      in_specs=in_specs,
      out_specs=out_specs,
      out_shape=out_shape,
  )(*args)


def _final_body(x_ref, agg_ref, deg_ref, ws_ref, wn_ref, b_ref, o_ref):
  aggv = agg_ref[0] + agg_ref[1]
  degv = deg_ref[0, :, 0:1] + deg_ref[1, :, 0:1]
  hn = aggv * (1.0 / (degv + 1.0))
  o_ref[...] = (
      jnp.dot(x_ref[...], ws_ref[...], preferred_element_type=jnp.float32)
      + jnp.dot(hn, wn_ref[...], preferred_element_type=jnp.float32)
      + b_ref[...])


def _final(x, agg, deg, w_self, w_neigh, b):
  dn = w_self.shape[1]
  return pl.pallas_call(
      _final_body,
      grid=(NN // _R,),
      in_specs=[
          pl.BlockSpec((_R, 128), lambda i: (i, 0)),
          pl.BlockSpec((2, _R, 128), lambda i: (0, i, 0)),
          pl.BlockSpec((2, _R, 128), lambda i: (0, i, 0)),
          pl.BlockSpec((128, dn), lambda i: (0, 0)),
          pl.BlockSpec((128, dn), lambda i: (0, 0)),
          pl.BlockSpec((1, dn), lambda i: (0, 0)),
      ],
      out_specs=pl.BlockSpec((_R, dn), lambda i: (i, 0)),
      out_shape=jax.ShapeDtypeStruct((NN, dn), jnp.float32),
  )(x, agg, deg, w_self, w_neigh, b)


def kernel(features, edge_index, W_self1, W_neigh1, b1,
           W_self2, W_neigh2, b2, W_self3, W_neigh3, b3):
  src = edge_index[0].astype(jnp.int32)
  dst = edge_index[1].astype(jnp.int32)
  npad = EPAD - EE
  srcp = jnp.concatenate([src, jnp.zeros((npad,), jnp.int32)]).reshape(
      TOTCH, CH)
  dstp = jnp.concatenate([dst, jnp.full((npad,), NN, jnp.int32)]).reshape(
      TOTCH, CH)
  dstp3 = dstp.reshape(NW, NCH, CH)
  z128 = jnp.zeros((STRIPE, 128), jnp.float32)
  ones = jnp.ones((CH, 128), jnp.float32)

  deg, = _sc_deg(dstp3, z128, ones)
  p1 = _mm(features, W_neigh1)
  agg1, = _sc_agg(p1, srcp, dstp, z128)
  h1, p2 = _combine(features, agg1, deg, W_self1,
                    b1.reshape(1, -1), W_neigh2)
  agg2, = _sc_agg(p2, srcp, dstp, z128)
  h2, = _combine(h1, agg2, deg, W_self2, b2.reshape(1, -1))
  agg3, = _sc_agg(h2, srcp, dstp, z128)
  return _final(h2, agg3, deg, W_self3, W_neigh3, b3.reshape(1, -1))


# speedup vs baseline: 1.1381x; 1.1063x over previous
import jax
import jax.numpy as jnp
from jax import lax
from jax.experimental import pallas as pl
from jax.experimental.pallas import tpu as pltpu
from jax.experimental.pallas import tpu_sc as plsc

NN = 10000
EE = 320000
NW = 32
CH = 128
NCH = 80
TOTCH = NW * NCH
BCH = 8
CH0 = 112
CH1 = 48
EPAD = TOTCH * CH
NPAD = 10112
STRIPE = NPAD // 16
DEG_BCH = 16
DEG_NBLK = NCH // DEG_BCH

_SC_MESH = plsc.VectorSubcoreMesh(core_axis_name="c", subcore_axis_name="s")


def _sc_agg_body(p_hbm, src_hbm, dst_hbm, z_hbm, agg_out,
                 idx_s, idx_d, rows, acc_sh, sem0, sem1):
  c = lax.axis_index("c")
  s = lax.axis_index("s")
  sems = (sem0, sem1)
  pltpu.sync_copy(z_hbm, acc_sh.at[pl.ds(s * STRIPE, STRIPE)])
  plsc.subcore_barrier()
  base = CH0 * s + c * (16 * CH0 + (CH1 - CH0) * s)
  nblk = (CH0 // BCH) + c * ((CH1 - CH0) // BCH)

  def start(j, buf):
    pltpu.async_copy(p_hbm.at[idx_s.at[j]], rows.at[buf], sems[buf])

  def finish(j, buf):
    pltpu.make_async_copy(p_hbm.at[idx_s.at[j]], rows.at[buf],
                          sems[buf]).wait()
    pltpu.sync_copy(rows.at[buf], acc_sh.at[idx_d.at[j]], add=True)

  def blk(b, carry):
    r0 = pl.multiple_of(base + b * BCH, 8)
    pltpu.sync_copy(src_hbm.at[pl.ds(r0, BCH)], idx_s)
    pltpu.sync_copy(dst_hbm.at[pl.ds(r0, BCH)], idx_d)
    start(0, 0)

    def pair(k, carry2):
      j0 = 2 * k
      start(j0 + 1, 1)
      finish(j0, 0)

      @pl.when(j0 + 2 < BCH)
      def _():
        start(j0 + 2, 0)

      finish(j0 + 1, 1)
      return carry2

    lax.fori_loop(0, BCH // 2, pair, 0)
    return carry

  lax.fori_loop(0, nblk, blk, 0)
  plsc.subcore_barrier()
  pltpu.sync_copy(acc_sh.at[pl.ds(s * STRIPE, STRIPE)],
                  agg_out.at[c, pl.ds(s * STRIPE, STRIPE)])


_sc_agg = pl.kernel(
    _sc_agg_body,
    out_type=[jax.ShapeDtypeStruct((2, NPAD, 128), jnp.float32)],
    mesh=_SC_MESH,
    scratch_types=[
        pltpu.VMEM((BCH, CH), jnp.int32),
        pltpu.VMEM((BCH, CH), jnp.int32),
        pltpu.VMEM((2, CH, 128), jnp.float32),
        pltpu.VMEM_SHARED((NPAD, 128), jnp.float32),
        pltpu.SemaphoreType.DMA,
        pltpu.SemaphoreType.DMA,
    ])


def _sc_deg_body(dst_hbm, z_hbm, ones_hbm, deg_out, idx_d, ones_v, acc_sh):
  c = lax.axis_index("c")
  s = lax.axis_index("s")
  w = c * 16 + s
  pltpu.sync_copy(z_hbm, acc_sh.at[pl.ds(s * STRIPE, STRIPE)])
  pltpu.sync_copy(ones_hbm, ones_v)
  plsc.subcore_barrier()

  def blk(b, carry):
    pltpu.sync_copy(dst_hbm.at[w, pl.ds(b * DEG_BCH, DEG_BCH)], idx_d)

    def chunk(j, carry2):
      pltpu.sync_copy(ones_v, acc_sh.at[idx_d.at[j]], add=True)
      return carry2

    lax.fori_loop(0, DEG_BCH, chunk, 0)
    return carry

  lax.fori_loop(0, DEG_NBLK, blk, 0)
  plsc.subcore_barrier()
  pltpu.sync_copy(acc_sh.at[pl.ds(s * STRIPE, STRIPE)],
                  deg_out.at[c, pl.ds(s * STRIPE, STRIPE)])


_sc_deg = pl.kernel(
    _sc_deg_body,
    out_type=[jax.ShapeDtypeStruct((2, NPAD, 128), jnp.float32)],
    mesh=_SC_MESH,
    scratch_types=[
        pltpu.VMEM((DEG_BCH, CH), jnp.int32),
        pltpu.VMEM((CH, 128), jnp.float32),
        pltpu.VMEM_SHARED((NPAD, 128), jnp.float32),
    ])

_R = 400


def _mm_body(x_ref, w_ref, o_ref):
  o_ref[...] = jnp.dot(x_ref[...], w_ref[...],
                       preferred_element_type=jnp.float32)


def _mm(x, w):
  n, k = x.shape
  m = w.shape[1]
  return pl.pallas_call(
      _mm_body,
      grid=(n // _R,),
      in_specs=[pl.BlockSpec((_R, k), lambda i: (i, 0)),
                pl.BlockSpec((k, m), lambda i: (0, 0))],
      out_specs=pl.BlockSpec((_R, m), lambda i: (i, 0)),
      out_shape=jax.ShapeDtypeStruct((n, m), jnp.float32),
  )(x, w)


def _combine_body(x_ref, agg_ref, deg_ref, ws_ref, b_ref, wn_ref,
                  h_ref, p_ref):
  aggv = agg_ref[0] + agg_ref[1]
  degv = deg_ref[0, :, 0:1] + deg_ref[1, :, 0:1]
  r = 1.0 / (degv + 1.0)
  h = jnp.maximum(
      jnp.dot(x_ref[...], ws_ref[...], preferred_element_type=jnp.float32)
      + aggv * r + b_ref[...], 0.0)
  h_ref[...] = h
  if p_ref is not None:
    p_ref[...] = jnp.dot(h, wn_ref[...], preferred_element_type=jnp.float32)


def _combine(x, agg, deg, w_self, b, w_next=None):
  in_specs = [
      pl.BlockSpec((_R, 128), lambda i: (i, 0)),
      pl.BlockSpec((2, _R, 128), lambda i: (0, i, 0)),
      pl.BlockSpec((2, _R, 128), lambda i: (0, i, 0)),
      pl.BlockSpec((128, 128), lambda i: (0, 0)),
      pl.BlockSpec((1, 128), lambda i: (0, 0)),
  ]
  out_specs = [pl.BlockSpec((_R, 128), lambda i: (i, 0))]
  out_shape = [jax.ShapeDtypeStruct((NN, 128), jnp.float32)]
  args = [x, agg, deg, w_self, b]
  if w_next is not None:
    dn = w_next.shape[1]
    in_specs.append(pl.BlockSpec((128, dn), lambda i: (0, 0)))
    out_specs.append(pl.BlockSpec((_R, dn), lambda i: (i, 0)))
    out_shape.append(jax.ShapeDtypeStruct((NN, dn), jnp.float32))
    args.append(w_next)
    body = _combine_body
  else:
    body = lambda x_ref, agg_ref, deg_ref, ws_ref, b_ref, h_ref: (
        _combine_body(x_ref, agg_ref, deg_ref, ws_ref, b_ref, None,
                      h_ref, None))
  return pl.pallas_call(
      body,
      grid=(NN // _R,),
      in_specs=in_specs,
      out_specs=out_specs,
      out_shape=out_shape,
  )(*args)


def _final_body(x_ref, agg_ref, deg_ref, ws_ref, wn_ref, b_ref, o_ref):
  aggv = agg_ref[0] + agg_ref[1]
  degv = deg_ref[0, :, 0:1] + deg_ref[1, :, 0:1]
  hn = aggv * (1.0 / (degv + 1.0))
  o_ref[...] = (
      jnp.dot(x_ref[...], ws_ref[...], preferred_element_type=jnp.float32)
      + jnp.dot(hn, wn_ref[...], preferred_element_type=jnp.float32)
      + b_ref[...])


def _final(x, agg, deg, w_self, w_neigh, b):
  dn = w_self.shape[1]
  return pl.pallas_call(
      _final_body,
      grid=(NN // _R,),
      in_specs=[
          pl.BlockSpec((_R, 128), lambda i: (i, 0)),
          pl.BlockSpec((2, _R, 128), lambda i: (0, i, 0)),
          pl.BlockSpec((2, _R, 128), lambda i: (0, i, 0)),
          pl.BlockSpec((128, dn), lambda i: (0, 0)),
          pl.BlockSpec((128, dn), lambda i: (0, 0)),
          pl.BlockSpec((1, dn), lambda i: (0, 0)),
      ],
      out_specs=pl.BlockSpec((_R, dn), lambda i: (i, 0)),
      out_shape=jax.ShapeDtypeStruct((NN, dn), jnp.float32),
  )(x, agg, deg, w_self, w_neigh, b)


def kernel(features, edge_index, W_self1, W_neigh1, b1,
           W_self2, W_neigh2, b2, W_self3, W_neigh3, b3):
  src = edge_index[0].astype(jnp.int32)
  dst = edge_index[1].astype(jnp.int32)
  npad = EPAD - EE
  srcp = jnp.concatenate([src, jnp.zeros((npad,), jnp.int32)]).reshape(
      TOTCH, CH)
  dstp = jnp.concatenate([dst, jnp.full((npad,), NN, jnp.int32)]).reshape(
      TOTCH, CH)
  dstp3 = dstp.reshape(NW, NCH, CH)
  z128 = jnp.zeros((STRIPE, 128), jnp.float32)
  ones = jnp.ones((CH, 128), jnp.float32)

  deg, = _sc_deg(dstp3, z128, ones)
  p1 = _mm(features, W_neigh1)
  agg1, = _sc_agg(p1, srcp, dstp, z128)
  h1, p2 = _combine(features, agg1, deg, W_self1,
                    b1.reshape(1, -1), W_neigh2)
  agg2, = _sc_agg(p2, srcp, dstp, z128)
  h2, = _combine(h1, agg2, deg, W_self2, b2.reshape(1, -1))
  agg3, = _sc_agg(h2, srcp, dstp, z128)
  return _final(h2, agg3, deg, W_self3, W_neigh3, b3.reshape(1, -1))


# speedup vs baseline: 1.1721x; 1.0299x over previous
import jax
import jax.numpy as jnp
from jax import lax
from jax.experimental import pallas as pl
from jax.experimental.pallas import tpu as pltpu
from jax.experimental.pallas import tpu_sc as plsc

NN = 10000
EE = 320000
NW = 32
CH = 128
NCH = 80
TOTCH = NW * NCH
BCH = 8
CH0 = 136
CH1 = 24
EPAD = TOTCH * CH
NPAD = 10112
STRIPE = NPAD // 16
DEG_BCH = 16
DEG_NBLK = NCH // DEG_BCH

_SC_MESH = plsc.VectorSubcoreMesh(core_axis_name="c", subcore_axis_name="s")


def _sc_agg_body(p_hbm, src_hbm, dst_hbm, z_hbm, agg_out,
                 idx_s, idx_d, rows, acc_sh, sem0, sem1):
  c = lax.axis_index("c")
  s = lax.axis_index("s")
  sems = (sem0, sem1)
  pltpu.sync_copy(z_hbm, acc_sh.at[pl.ds(s * STRIPE, STRIPE)])
  plsc.subcore_barrier()
  base = CH0 * s + c * (16 * CH0 + (CH1 - CH0) * s)
  nblk = (CH0 // BCH) + c * ((CH1 - CH0) // BCH)

  def start(j, buf):
    pltpu.async_copy(p_hbm.at[idx_s.at[j]], rows.at[buf], sems[buf])

  def finish(j, buf):
    pltpu.make_async_copy(p_hbm.at[idx_s.at[j]], rows.at[buf],
                          sems[buf]).wait()
    pltpu.sync_copy(rows.at[buf], acc_sh.at[idx_d.at[j]], add=True)

  def blk(b, carry):
    r0 = pl.multiple_of(base + b * BCH, 8)
    pltpu.sync_copy(src_hbm.at[pl.ds(r0, BCH)], idx_s)
    pltpu.sync_copy(dst_hbm.at[pl.ds(r0, BCH)], idx_d)
    start(0, 0)

    def pair(k, carry2):
      j0 = 2 * k
      start(j0 + 1, 1)
      finish(j0, 0)

      @pl.when(j0 + 2 < BCH)
      def _():
        start(j0 + 2, 0)

      finish(j0 + 1, 1)
      return carry2

    lax.fori_loop(0, BCH // 2, pair, 0)
    return carry

  lax.fori_loop(0, nblk, blk, 0)
  plsc.subcore_barrier()
  pltpu.sync_copy(acc_sh.at[pl.ds(s * STRIPE, STRIPE)],
                  agg_out.at[c, pl.ds(s * STRIPE, STRIPE)])


_sc_agg = pl.kernel(
    _sc_agg_body,
    out_type=[jax.ShapeDtypeStruct((2, NPAD, 128), jnp.float32)],
    mesh=_SC_MESH,
    scratch_types=[
        pltpu.VMEM((BCH, CH), jnp.int32),
        pltpu.VMEM((BCH, CH), jnp.int32),
        pltpu.VMEM((2, CH, 128), jnp.float32),
        pltpu.VMEM_SHARED((NPAD, 128), jnp.float32),
        pltpu.SemaphoreType.DMA,
        pltpu.SemaphoreType.DMA,
    ])


def _sc_deg_body(dst_hbm, z_hbm, ones_hbm, deg_out, idx_d, ones_v, acc_sh):
  c = lax.axis_index("c")
  s = lax.axis_index("s")
  w = c * 16 + s
  pltpu.sync_copy(z_hbm, acc_sh.at[pl.ds(s * STRIPE, STRIPE)])
  pltpu.sync_copy(ones_hbm, ones_v)
  plsc.subcore_barrier()

  def blk(b, carry):
    pltpu.sync_copy(dst_hbm.at[w, pl.ds(b * DEG_BCH, DEG_BCH)], idx_d)

    def chunk(j, carry2):
      pltpu.sync_copy(ones_v, acc_sh.at[idx_d.at[j]], add=True)
      return carry2

    lax.fori_loop(0, DEG_BCH, chunk, 0)
    return carry

  lax.fori_loop(0, DEG_NBLK, blk, 0)
  plsc.subcore_barrier()
  pltpu.sync_copy(acc_sh.at[pl.ds(s * STRIPE, STRIPE)],
                  deg_out.at[c, pl.ds(s * STRIPE, STRIPE)])


_sc_deg = pl.kernel(
    _sc_deg_body,
    out_type=[jax.ShapeDtypeStruct((2, NPAD, 128), jnp.float32)],
    mesh=_SC_MESH,
    scratch_types=[
        pltpu.VMEM((DEG_BCH, CH), jnp.int32),
        pltpu.VMEM((CH, 128), jnp.float32),
        pltpu.VMEM_SHARED((NPAD, 128), jnp.float32),
    ])

_R = 400


def _mm_body(x_ref, w_ref, o_ref):
  o_ref[...] = jnp.dot(x_ref[...], w_ref[...],
                       preferred_element_type=jnp.float32)


def _mm(x, w):
  n, k = x.shape
  m = w.shape[1]
  return pl.pallas_call(
      _mm_body,
      grid=(n // _R,),
      in_specs=[pl.BlockSpec((_R, k), lambda i: (i, 0)),
                pl.BlockSpec((k, m), lambda i: (0, 0))],
      out_specs=pl.BlockSpec((_R, m), lambda i: (i, 0)),
      out_shape=jax.ShapeDtypeStruct((n, m), jnp.float32),
  )(x, w)


def _combine_body(x_ref, agg_ref, deg_ref, ws_ref, b_ref, wn_ref,
                  h_ref, p_ref):
  aggv = agg_ref[0] + agg_ref[1]
  degv = deg_ref[0, :, 0:1] + deg_ref[1, :, 0:1]
  r = 1.0 / (degv + 1.0)
  h = jnp.maximum(
      jnp.dot(x_ref[...], ws_ref[...], preferred_element_type=jnp.float32)
      + aggv * r + b_ref[...], 0.0)
  h_ref[...] = h
  if p_ref is not None:
    p_ref[...] = jnp.dot(h, wn_ref[...], preferred_element_type=jnp.float32)


def _combine(x, agg, deg, w_self, b, w_next=None):
  in_specs = [
      pl.BlockSpec((_R, 128), lambda i: (i, 0)),
      pl.BlockSpec((2, _R, 128), lambda i: (0, i, 0)),
      pl.BlockSpec((2, _R, 128), lambda i: (0, i, 0)),
      pl.BlockSpec((128, 128), lambda i: (0, 0)),
      pl.BlockSpec((1, 128), lambda i: (0, 0)),
  ]
  out_specs = [pl.BlockSpec((_R, 128), lambda i: (i, 0))]
  out_shape = [jax.ShapeDtypeStruct((NN, 128), jnp.float32)]
  args = [x, agg, deg, w_self, b]
  if w_next is not None:
    dn = w_next.shape[1]
    in_specs.append(pl.BlockSpec((128, dn), lambda i: (0, 0)))
    out_specs.append(pl.BlockSpec((_R, dn), lambda i: (i, 0)))
    out_shape.append(jax.ShapeDtypeStruct((NN, dn), jnp.float32))
    args.append(w_next)
    body = _combine_body
  else:
    body = lambda x_ref, agg_ref, deg_ref, ws_ref, b_ref, h_ref: (
        _combine_body(x_ref, agg_ref, deg_ref, ws_ref, b_ref, None,
                      h_ref, None))
  return pl.pallas_call(
      body,
      grid=(NN // _R,),
      in_specs=in_specs,
      out_specs=out_specs,
      out_shape=out_shape,
  )(*args)


def _final_body(x_ref, agg_ref, deg_ref, ws_ref, wn_ref, b_ref, o_ref):
  aggv = agg_ref[0] + agg_ref[1]
  degv = deg_ref[0, :, 0:1] + deg_ref[1, :, 0:1]
  hn = aggv * (1.0 / (degv + 1.0))
  o_ref[...] = (
      jnp.dot(x_ref[...], ws_ref[...], preferred_element_type=jnp.float32)
      + jnp.dot(hn, wn_ref[...], preferred_element_type=jnp.float32)
      + b_ref[...])


def _final(x, agg, deg, w_self, w_neigh, b):
  dn = w_self.shape[1]
  return pl.pallas_call(
      _final_body,
      grid=(NN // _R,),
      in_specs=[
          pl.BlockSpec((_R, 128), lambda i: (i, 0)),
          pl.BlockSpec((2, _R, 128), lambda i: (0, i, 0)),
          pl.BlockSpec((2, _R, 128), lambda i: (0, i, 0)),
          pl.BlockSpec((128, dn), lambda i: (0, 0)),
          pl.BlockSpec((128, dn), lambda i: (0, 0)),
          pl.BlockSpec((1, dn), lambda i: (0, 0)),
      ],
      out_specs=pl.BlockSpec((_R, dn), lambda i: (i, 0)),
      out_shape=jax.ShapeDtypeStruct((NN, dn), jnp.float32),
  )(x, agg, deg, w_self, w_neigh, b)


def kernel(features, edge_index, W_self1, W_neigh1, b1,
           W_self2, W_neigh2, b2, W_self3, W_neigh3, b3):
  src = edge_index[0].astype(jnp.int32)
  dst = edge_index[1].astype(jnp.int32)
  npad = EPAD - EE
  srcp = jnp.concatenate([src, jnp.zeros((npad,), jnp.int32)]).reshape(
      TOTCH, CH)
  dstp = jnp.concatenate([dst, jnp.full((npad,), NN, jnp.int32)]).reshape(
      TOTCH, CH)
  dstp3 = dstp.reshape(NW, NCH, CH)
  z128 = jnp.zeros((STRIPE, 128), jnp.float32)
  ones = jnp.ones((CH, 128), jnp.float32)

  deg, = _sc_deg(dstp3, z128, ones)
  p1 = _mm(features, W_neigh1)
  agg1, = _sc_agg(p1, srcp, dstp, z128)
  h1, p2 = _combine(features, agg1, deg, W_self1,
                    b1.reshape(1, -1), W_neigh2)
  agg2, = _sc_agg(p2, srcp, dstp, z128)
  h2, = _combine(h1, agg2, deg, W_self2, b2.reshape(1, -1))
  agg3, = _sc_agg(h2, srcp, dstp, z128)
  return _final(h2, agg3, deg, W_self3, W_neigh3, b3.reshape(1, -1))
